# Initial kernel scaffold; baseline (speedup 1.0000x reference)
import jax
import jax.numpy as jnp
from jax.experimental import pallas as pl


def kernel(probs, index, latent):
    raise NotImplementedError("write your pallas kernel here")


# speedup vs baseline: 4.1818x; 4.1818x over previous
import functools

import jax
import jax.numpy as jnp
from jax import lax
from jax.experimental import pallas as pl
from jax.experimental.pallas import tpu as pltpu
from jax.experimental.pallas import tpu_sc as plsc

N = 100000
C = 128
B = 16384
NC = 2
NS = 16
NW = NC * NS
RANGE = 3128
NPAD = RANGE * NW
CHUNK = B // NW
SUB = 128
NSUB = CHUNK // SUB

_mesh = functools.partial(
    plsc.VectorSubcoreMesh, core_axis_name="c", subcore_axis_name="s"
)


@functools.partial(
    pl.kernel,
    out_type=jax.ShapeDtypeStruct((NPAD,), jnp.int32),
    mesh=_mesh(),
    compiler_params=pltpu.CompilerParams(needs_layout_passes=False),
    scratch_types=[
        pltpu.VMEM((B,), jnp.int32),
        pltpu.VMEM((RANGE,), jnp.int32),
    ],
)
def _winner_kernel(idx_hbm, win_hbm, idx_v, win_v):
    wid = lax.axis_index("s") * NC + lax.axis_index("c")
    lo = wid * RANGE
    pltpu.sync_copy(idx_hbm, idx_v)

    def body(j, carry):
        v = idx_v[pl.ds(j * 16, 16)]
        rel = v - lo
        mask = (rel >= 0) & (rel < RANGE)
        rel = jnp.clip(rel, 0, RANGE - 1)
        b = lax.iota(jnp.int32, 16) + j * 16
        plsc.store_scatter(win_v, [rel], b, mask=mask)
        return carry

    lax.fori_loop(0, B // 16, body, 0)
    pltpu.sync_copy(win_v, win_hbm.at[pl.ds(lo, RANGE)])


@functools.partial(
    pl.kernel,
    out_type=jax.ShapeDtypeStruct((B, C), jnp.float32),
    mesh=_mesh(),
    compiler_params=pltpu.CompilerParams(needs_layout_passes=False),
    scratch_types=[
        pltpu.VMEM((CHUNK,), jnp.int32),
        pltpu.VMEM((CHUNK,), jnp.int32),
        pltpu.VMEM((SUB, C), jnp.float32),
        pltpu.VMEM((SUB, C), jnp.float32),
        pltpu.VMEM((SUB, C), jnp.float32),
        pltpu.SemaphoreType.DMA,
    ],
)
def _main_kernel(probs_hbm, idx_hbm, latent_hbm, win_hbm, out_hbm,
                 idx_v, w_v, p_v, l_v, o_v, sem):
    wid = lax.axis_index("s") * NC + lax.axis_index("c")
    base = wid * CHUNK
    pltpu.sync_copy(idx_hbm.at[pl.ds(base, CHUNK)], idx_v)

    for s in range(NSUB):
        idx_s = idx_v.at[pl.ds(s * SUB, SUB)]
        w_s = w_v.at[pl.ds(s * SUB, SUB)]
        pltpu.async_copy(win_hbm.at[idx_s], w_s, sem).wait()
        cp_p = pltpu.async_copy(probs_hbm.at[w_s], p_v, sem)
        cp_l = pltpu.async_copy(latent_hbm.at[idx_s], l_v, sem)
        cp_p.wait()
        cp_l.wait()

        def row(r, carry):
            segs = []
            tot = jnp.zeros((16,), jnp.float32)
            for j in range(C // 16):
                v = p_v[r, pl.ds(j * 16, 16)]
                v = jnp.clip(v, 0.0001, 1.0 - 0.0001)
                segs.append(v)
                tot = tot + v
            rs1 = jnp.full((16,), 1.0, jnp.float32) / jnp.broadcast_to(
                jnp.sum(tot), (16,))
            tot2 = jnp.zeros((16,), jnp.float32)
            sq = []
            for j in range(C // 16):
                g = l_v[r, pl.ds(j * 16, 16)]
                nr = 0.9 * g + (1.0 - 0.9) * (segs[j] * rs1)
                q = nr * nr
                sq.append(q)
                tot2 = tot2 + q
            rs2 = jnp.full((16,), 1.0, jnp.float32) / jnp.broadcast_to(
                jnp.sum(tot2), (16,))
            for j in range(C // 16):
                o_v[r, pl.ds(j * 16, 16)] = sq[j] * rs2
            return carry

        lax.fori_loop(0, SUB, row, 0)
        pltpu.sync_copy(o_v, out_hbm.at[pl.ds(base + s * SUB, SUB)])


def kernel(probs, index, latent):
    win = _winner_kernel(index)
    return _main_kernel(probs, index, latent, win)


# speedup vs baseline: 4.9584x; 1.1857x over previous
import functools

import jax
import jax.numpy as jnp
from jax import lax
from jax.experimental import pallas as pl
from jax.experimental.pallas import tpu as pltpu
from jax.experimental.pallas import tpu_sc as plsc

N = 100000
C = 128
B = 16384
NC = 2
NS = 16
NW = NC * NS
RANGE = 6256
NPAD = RANGE * NS
CHUNK = B // NW
SUB = 128
NSUB = CHUNK // SUB


@functools.partial(
    pl.kernel,
    out_type=jax.ShapeDtypeStruct((B, C), jnp.float32),
    mesh=plsc.VectorSubcoreMesh(core_axis_name="c", subcore_axis_name="s"),
    compiler_params=pltpu.CompilerParams(needs_layout_passes=False),
    scratch_types=[
        pltpu.VMEM((B,), jnp.int32),
        pltpu.VMEM((RANGE,), jnp.int32),
        pltpu.VMEM_SHARED((NPAD,), jnp.int32),
        pltpu.VMEM((CHUNK,), jnp.int32),
        pltpu.VMEM((CHUNK, C), jnp.float32),
        pltpu.VMEM((SUB, C), jnp.float32),
        pltpu.VMEM((SUB, C), jnp.float32),
        pltpu.SemaphoreType.DMA,
        pltpu.SemaphoreType.DMA,
    ],
)
def _sc_kernel(probs_hbm, idx_hbm, latent_hbm, out_hbm,
               idx_v, win_v, table_s, w_v, l_v, p_v, o_v, sem, sem_l):
    sid = lax.axis_index("s")
    wid = sid * NC + lax.axis_index("c")
    base = wid * CHUNK
    lo = sid * RANGE

    pltpu.sync_copy(idx_hbm, idx_v)
    my_idx = idx_v.at[pl.ds(base, CHUNK)]
    cp_lat = pltpu.async_copy(latent_hbm.at[my_idx], l_v, sem_l)

    def body(j, carry):
        v = idx_v[pl.ds(j * 16, 16)]
        rel = v - lo
        mask = (rel >= 0) & (rel < RANGE)
        rel = jnp.clip(rel, 0, RANGE - 1)
        b = lax.iota(jnp.int32, 16) + j * 16
        plsc.store_scatter(win_v, [rel], b, mask=mask)
        return carry

    lax.fori_loop(0, B // 16, body, 0)
    pltpu.sync_copy(win_v, table_s.at[pl.ds(lo, RANGE)])
    plsc.subcore_barrier()

    pltpu.async_copy(table_s.at[my_idx], w_v, sem).wait()
    cp_lat.wait()

    for s in range(NSUB):
        w_s = w_v.at[pl.ds(s * SUB, SUB)]
        pltpu.async_copy(probs_hbm.at[w_s], p_v, sem).wait()

        def row(r, carry):
            segs = []
            tot = jnp.zeros((16,), jnp.float32)
            for j in range(C // 16):
                v = p_v[r, pl.ds(j * 16, 16)]
                v = jnp.clip(v, 0.0001, 1.0 - 0.0001)
                segs.append(v)
                tot = tot + v
            rs1 = jnp.full((16,), 1.0, jnp.float32) / jnp.broadcast_to(
                jnp.sum(tot), (16,))
            tot2 = jnp.zeros((16,), jnp.float32)
            sq = []
            for j in range(C // 16):
                g = l_v[s * SUB + r, pl.ds(j * 16, 16)]
                nr = 0.9 * g + (1.0 - 0.9) * (segs[j] * rs1)
                q = nr * nr
                sq.append(q)
                tot2 = tot2 + q
            rs2 = jnp.full((16,), 1.0, jnp.float32) / jnp.broadcast_to(
                jnp.sum(tot2), (16,))
            for j in range(C // 16):
                o_v[r, pl.ds(j * 16, 16)] = sq[j] * rs2
            return carry

        lax.fori_loop(0, SUB, row, 0)
        pltpu.sync_copy(o_v, out_hbm.at[pl.ds(base + s * SUB, SUB)])


def kernel(probs, index, latent):
    return _sc_kernel(probs, index, latent)
